# initial kernel scaffold (unmeasured)
import jax
import jax.numpy as jnp
from jax import lax
from jax.experimental import pallas as pl
from jax.experimental.pallas import tpu as pltpu

N_DEV = 4
SQ = 1024
SKV_PER = 1024
SKV = N_DEV * SKV_PER
HQ_PER = 8
DH = 128
DM = 1024
HD_PER = HQ_PER * DH
SCALE = 0.08838834764831843
LOCAL_W = 128
N_GLOB = 32
KV_LOC = 2 * SKV_PER


def kernel(x, Wq, K_ext, V_ext, Wo):
    x2 = x.reshape(SQ, DM)
    K2 = K_ext.reshape(SKV_PER, 32 * DH)
    V2 = V_ext.reshape(SKV_PER, 32 * DH)

    def body(x_ref, wq_ref, k_ref, v_ref, wo_ref, out_ref,
             kall, vall, comm,
             k_send, v_send, k_recv, v_recv, ring_send, ring_recv):
        my = lax.axis_index("i")

        barrier_sem = pltpu.get_barrier_semaphore()
        for d in range(1, N_DEV):
            peer = lax.rem(my + d, N_DEV)
            pl.semaphore_signal(
                barrier_sem, inc=1,
                device_id=(peer,), device_id_type=pl.DeviceIdType.MESH,
            )
        pl.semaphore_wait(barrier_sem, N_DEV - 1)

        sends = []
        for d in range(1, N_DEV):
            t = lax.rem(my + d, N_DEV)
            for (src_ref, dst, ssem, rsem) in (
                (k_ref, kall, k_send, k_recv),
                (v_ref, vall, v_send, v_recv),
            ):
                rdma = pltpu.make_async_remote_copy(
                    src_ref=src_ref.at[:, pl.ds(t * HD_PER, HD_PER)],
                    dst_ref=dst.at[pl.ds(my * SKV_PER, SKV_PER), :],
                    send_sem=ssem.at[d - 1],
                    recv_sem=rsem.at[d - 1],
                    device_id=(t,),
                    device_id_type=pl.DeviceIdType.MESH,
                )
                rdma.start()
                sends.append(rdma)

        kall[pl.ds(my * SKV_PER, SKV_PER), :] = k_ref[:, pl.ds(my * HD_PER, HD_PER)]
        vall[pl.ds(my * SKV_PER, SKV_PER), :] = v_ref[:, pl.ds(my * HD_PER, HD_PER)]

        q = jnp.dot(x_ref[:, :], wq_ref[:, :], preferred_element_type=jnp.float32)

        for rdma in sends:
            rdma.wait_send()
        for d in range(1, N_DEV):
            src = lax.rem(my - d + N_DEV, N_DEV)
            for (src_ref, dst, ssem, rsem) in (
                (k_ref, kall, k_send, k_recv),
                (v_ref, vall, v_send, v_recv),
            ):
                recv = pltpu.make_async_remote_copy(
                    src_ref=src_ref.at[:, pl.ds(0, HD_PER)],
                    dst_ref=dst.at[pl.ds(src * SKV_PER, SKV_PER), :],
                    send_sem=ssem.at[d - 1],
                    recv_sem=rsem.at[d - 1],
                    device_id=(src,),
                    device_id_type=pl.DeviceIdType.MESH,
                )
                recv.wait_recv()

        n_loc = SQ - N_GLOB
        qi_l = lax.broadcasted_iota(jnp.int32, (n_loc, KV_LOC), 0) + N_GLOB
        ki_l = lax.broadcasted_iota(jnp.int32, (n_loc, KV_LOC), 1)
        mask_l = (jnp.abs(qi_l - ki_l) <= LOCAL_W) | (ki_l < N_GLOB)

        partial = jnp.zeros((SQ, DM), jnp.float32)
        for h in range(HQ_PER):
            c0 = h * DH
            qh = q[:, c0:c0 + DH]
            kh = kall[:, c0:c0 + DH]
            vh = vall[:, c0:c0 + DH]

            qg = qh[:N_GLOB, :]
            sg = lax.dot_general(qg, kh, (((1,), (1,)), ((), ()))) * SCALE
            sg = sg - jnp.max(sg, axis=1, keepdims=True)
            wg = jnp.exp(sg)
            wg = wg / jnp.sum(wg, axis=1, keepdims=True)
            ctx_g = jnp.dot(wg, vh, preferred_element_type=jnp.float32)

            ql = qh[N_GLOB:, :]
            sl = lax.dot_general(ql, kh[:KV_LOC, :], (((1,), (1,)), ((), ()))) * SCALE
            sl = jnp.where(mask_l, sl, -1e9)
            sl = sl - jnp.max(sl, axis=1, keepdims=True)
            wl = jnp.exp(sl)
            wl = wl / jnp.sum(wl, axis=1, keepdims=True)
            ctx_l = jnp.dot(wl, vh[:KV_LOC, :], preferred_element_type=jnp.float32)

            ctx_h = jnp.concatenate([ctx_g, ctx_l], axis=0)
            wo_h = wo_ref[c0:c0 + DH, :]
            partial = partial + jnp.dot(ctx_h, wo_h, preferred_element_type=jnp.float32)

        right = lax.rem(my + 1, N_DEV)
        comm[0, :, :] = partial
        acc = partial
        for h in range(N_DEV - 1):
            rdma = pltpu.make_async_remote_copy(
                src_ref=comm.at[h],
                dst_ref=comm.at[h + 1],
                send_sem=ring_send.at[h],
                recv_sem=ring_recv.at[h],
                device_id=(right,),
                device_id_type=pl.DeviceIdType.MESH,
            )
            rdma.start()
            rdma.wait()
            acc = acc + comm[h + 1, :, :]
        out_ref[:, :] = acc

    out = pl.pallas_call(
        body,
        out_shape=jax.ShapeDtypeStruct((SQ, DM), jnp.float32),
        in_specs=[pl.BlockSpec(memory_space=pltpu.VMEM)] * 5,
        out_specs=pl.BlockSpec(memory_space=pltpu.VMEM),
        scratch_shapes=[
            pltpu.VMEM((SKV, HD_PER), jnp.float32),
            pltpu.VMEM((SKV, HD_PER), jnp.float32),
            pltpu.VMEM((N_DEV, SQ, DM), jnp.float32),
            pltpu.SemaphoreType.DMA((N_DEV - 1,)),
            pltpu.SemaphoreType.DMA((N_DEV - 1,)),
            pltpu.SemaphoreType.DMA((N_DEV - 1,)),
            pltpu.SemaphoreType.DMA((N_DEV - 1,)),
            pltpu.SemaphoreType.DMA((N_DEV - 1,)),
            pltpu.SemaphoreType.DMA((N_DEV - 1,)),
        ],
        compiler_params=pltpu.CompilerParams(collective_id=0),
    )(x2, Wq, K2, V2, Wo)
    return out.reshape(1, SQ, DM)


# baseline (device time: 240224 ns/iter reference)
import jax
import jax.numpy as jnp
from jax import lax
from jax.experimental import pallas as pl
from jax.experimental.pallas import tpu as pltpu

N_DEV = 4
SQ = 1024
SKV_PER = 1024
SKV = N_DEV * SKV_PER
HQ_PER = 8
DH = 128
DM = 1024
HD_PER = HQ_PER * DH
SCALE = 0.08838834764831843
LOCAL_W = 128
N_GLOB = 32
KV_LOC = 2 * SKV_PER
LOC_TILE = 248
W_WIN = 512


def kernel(x, Wq, K_ext, V_ext, Wo):
    xb = x.reshape(SQ, DM).astype(jnp.bfloat16)
    Wqb = Wq.astype(jnp.bfloat16)
    Wob = Wo.astype(jnp.bfloat16)
    K2 = K_ext.reshape(SKV_PER, 32 * DH).astype(jnp.bfloat16)
    V2 = V_ext.reshape(SKV_PER, 32 * DH).astype(jnp.bfloat16)

    def body(x_ref, wq_ref, k_any, v_any, wo_ref, out_ref,
             kall, vall, qbuf, ctx_buf, comm,
             k_send, v_send, k_recv, v_recv, ring_send, ring_recv, loc_sem):
        my = lax.axis_index("i")

        barrier_sem = pltpu.get_barrier_semaphore()
        for d in range(1, N_DEV):
            peer = lax.rem(my + d, N_DEV)
            pl.semaphore_signal(
                barrier_sem, inc=1,
                device_id=(peer,), device_id_type=pl.DeviceIdType.MESH,
            )
        pl.semaphore_wait(barrier_sem, N_DEV - 1)

        sends = []
        for d in range(1, N_DEV):
            t = lax.rem(my + d, N_DEV)
            for (src_ref, dst, ssem, rsem) in (
                (k_any, kall, k_send, k_recv),
                (v_any, vall, v_send, v_recv),
            ):
                rdma = pltpu.make_async_remote_copy(
                    src_ref=src_ref.at[:, pl.ds(t * HD_PER, HD_PER)],
                    dst_ref=dst.at[pl.ds(my * SKV_PER, SKV_PER), :],
                    send_sem=ssem.at[d - 1],
                    recv_sem=rsem.at[d - 1],
                    device_id=(t,),
                    device_id_type=pl.DeviceIdType.MESH,
                )
                rdma.start()
                sends.append(rdma)

        lk = pltpu.make_async_copy(
            k_any.at[:, pl.ds(my * HD_PER, HD_PER)],
            kall.at[pl.ds(my * SKV_PER, SKV_PER), :],
            loc_sem.at[0],
        )
        lv = pltpu.make_async_copy(
            v_any.at[:, pl.ds(my * HD_PER, HD_PER)],
            vall.at[pl.ds(my * SKV_PER, SKV_PER), :],
            loc_sem.at[1],
        )
        lk.start()
        lv.start()

        q32 = jnp.dot(x_ref[:, :], wq_ref[:, :], preferred_element_type=jnp.float32)
        qbuf[:, :] = q32.astype(jnp.bfloat16)

        lk.wait()
        lv.wait()
        for rdma in sends:
            rdma.wait_send()
        for d in range(1, N_DEV):
            src = lax.rem(my - d + N_DEV, N_DEV)
            for (src_ref, dst, ssem, rsem) in (
                (k_any, kall, k_send, k_recv),
                (v_any, vall, v_send, v_recv),
            ):
                recv = pltpu.make_async_remote_copy(
                    src_ref=src_ref.at[:, pl.ds(0, HD_PER)],
                    dst_ref=dst.at[pl.ds(src * SKV_PER, SKV_PER), :],
                    send_sem=ssem.at[d - 1],
                    recv_sem=rsem.at[d - 1],
                    device_id=(src,),
                    device_id_type=pl.DeviceIdType.MESH,
                )
                recv.wait_recv()

        n_loc = SQ - N_GLOB

        def head_body(h, carry):
            c0 = h * DH
            qh = qbuf[:, pl.ds(c0, DH)]
            kh = kall[:, pl.ds(c0, DH)]
            vh = vall[:, pl.ds(c0, DH)]

            qg = qh[:N_GLOB, :]
            sg = lax.dot_general(
                qg, kh, (((1,), (1,)), ((), ())),
                preferred_element_type=jnp.float32) * SCALE
            sg = sg - jnp.max(sg, axis=1, keepdims=True)
            wg = jnp.exp(sg)
            den_g = jnp.sum(wg, axis=1, keepdims=True)
            ctx_g = jnp.dot(wg.astype(jnp.bfloat16), vh,
                            preferred_element_type=jnp.float32) / den_g
            ctx_buf[:N_GLOB, pl.ds(c0, DH)] = ctx_g.astype(jnp.bfloat16)

            k_blk = kh[:N_GLOB, :]
            v_blk = vh[:N_GLOB, :]

            def tile_body(t, c):
                r0 = N_GLOB + t * LOC_TILE
                w0 = 8 * jnp.maximum(0, 31 * t - 12)
                ql = qbuf[pl.ds(r0, LOC_TILE), pl.ds(c0, DH)]
                k_win = kall[pl.ds(w0, W_WIN), pl.ds(c0, DH)]
                v_win = vall[pl.ds(w0, W_WIN), pl.ds(c0, DH)]
                sw = lax.dot_general(
                    ql, k_win, (((1,), (1,)), ((), ())),
                    preferred_element_type=jnp.float32) * SCALE
                qi = lax.broadcasted_iota(jnp.int32, (LOC_TILE, W_WIN), 0) + r0
                ki = lax.broadcasted_iota(jnp.int32, (LOC_TILE, W_WIN), 1) + w0
                mask = (jnp.abs(qi - ki) <= LOCAL_W) | (ki < N_GLOB)
                sw = jnp.where(mask, sw, -1e9)
                sb = lax.dot_general(
                    ql, k_blk, (((1,), (1,)), ((), ())),
                    preferred_element_type=jnp.float32) * SCALE
                sb = jnp.where(t > 0, sb, -1e9)
                m = jnp.maximum(jnp.max(sw, axis=1, keepdims=True),
                                jnp.max(sb, axis=1, keepdims=True))
                ww = jnp.exp(sw - m)
                wb = jnp.exp(sb - m)
                den = (jnp.sum(ww, axis=1, keepdims=True)
                       + jnp.sum(wb, axis=1, keepdims=True))
                ctx_l = (jnp.dot(ww.astype(jnp.bfloat16), v_win,
                                 preferred_element_type=jnp.float32)
                         + jnp.dot(wb.astype(jnp.bfloat16), v_blk,
                                   preferred_element_type=jnp.float32)) / den
                ctx_buf[pl.ds(r0, LOC_TILE), pl.ds(c0, DH)] = (
                    ctx_l.astype(jnp.bfloat16))
                return c

            lax.fori_loop(0, n_loc // LOC_TILE, tile_body, 0)
            return carry

        lax.fori_loop(0, HQ_PER, head_body, 0)

        partial = jnp.dot(ctx_buf[:, :], wo_ref[:, :],
                          preferred_element_type=jnp.float32)

        right = lax.rem(my + 1, N_DEV)
        comm[0, :, :] = partial.astype(jnp.bfloat16)
        out_ref[:, :] = partial
        for h in range(N_DEV - 1):
            rdma = pltpu.make_async_remote_copy(
                src_ref=comm.at[h],
                dst_ref=comm.at[h + 1],
                send_sem=ring_send.at[h],
                recv_sem=ring_recv.at[h],
                device_id=(right,),
                device_id_type=pl.DeviceIdType.MESH,
            )
            rdma.start()
            rdma.wait()
            out_ref[:, :] = out_ref[:, :] + comm[h + 1, :, :].astype(jnp.float32)

    out = pl.pallas_call(
        body,
        out_shape=jax.ShapeDtypeStruct((SQ, DM), jnp.float32),
        in_specs=[
            pl.BlockSpec(memory_space=pltpu.VMEM),
            pl.BlockSpec(memory_space=pltpu.VMEM),
            pl.BlockSpec(memory_space=pltpu.MemorySpace.HBM),
            pl.BlockSpec(memory_space=pltpu.MemorySpace.HBM),
            pl.BlockSpec(memory_space=pltpu.VMEM),
        ],
        out_specs=pl.BlockSpec(memory_space=pltpu.VMEM),
        scratch_shapes=[
            pltpu.VMEM((SKV, HD_PER), jnp.bfloat16),
            pltpu.VMEM((SKV, HD_PER), jnp.bfloat16),
            pltpu.VMEM((SQ, HD_PER), jnp.bfloat16),
            pltpu.VMEM((SQ, HD_PER), jnp.bfloat16),
            pltpu.VMEM((N_DEV, SQ, DM), jnp.bfloat16),
            pltpu.SemaphoreType.DMA((N_DEV - 1,)),
            pltpu.SemaphoreType.DMA((N_DEV - 1,)),
            pltpu.SemaphoreType.DMA((N_DEV - 1,)),
            pltpu.SemaphoreType.DMA((N_DEV - 1,)),
            pltpu.SemaphoreType.DMA((N_DEV - 1,)),
            pltpu.SemaphoreType.DMA((N_DEV - 1,)),
            pltpu.SemaphoreType.DMA((2,)),
        ],
        compiler_params=pltpu.CompilerParams(collective_id=0),
    )(xb, Wqb, K2, V2, Wob)
    return out.reshape(1, SQ, DM)


# device time: 192966 ns/iter; 1.2449x vs baseline; 1.2449x over previous
import jax
import jax.numpy as jnp
from jax import lax
from jax.experimental import pallas as pl
from jax.experimental.pallas import tpu as pltpu

N_DEV = 4
SQ = 1024
SKV_PER = 1024
SKV = N_DEV * SKV_PER
HQ_PER = 8
DH = 128
DM = 1024
HD_PER = HQ_PER * DH
SCALE = 0.08838834764831843
LOCAL_W = 128
N_GLOB = 32
KV_LOC = 2 * SKV_PER
LOC_TILE = 248
W_WIN = 512
QROWS = SQ // N_DEV


def kernel(x, Wq, K_ext, V_ext, Wo):
    xb = x.reshape(SQ, DM).astype(jnp.bfloat16)
    Wqb = Wq.astype(jnp.bfloat16)
    Wob = Wo.astype(jnp.bfloat16)
    K2 = K_ext.reshape(SKV_PER, 32 * DH).astype(jnp.bfloat16)
    V2 = V_ext.reshape(SKV_PER, 32 * DH).astype(jnp.bfloat16)

    def body(x_ref, wq_ref, k_any, v_any, wo_ref, out_ref,
             kall, vall, qbuf, ctx_buf, pbuf, rs_buf, ag_buf,
             k_send, v_send, k_recv, v_recv,
             rs_send, rs_recv, ag_send, ag_recv, loc_sem):
        my = lax.axis_index("i")

        barrier_sem = pltpu.get_barrier_semaphore()
        for d in range(1, N_DEV):
            peer = lax.rem(my + d, N_DEV)
            pl.semaphore_signal(
                barrier_sem, inc=1,
                device_id=(peer,), device_id_type=pl.DeviceIdType.MESH,
            )
        pl.semaphore_wait(barrier_sem, N_DEV - 1)

        sends = []
        for d in range(1, N_DEV):
            t = lax.rem(my + d, N_DEV)
            for (src_ref, dst, ssem, rsem) in (
                (k_any, kall, k_send, k_recv),
                (v_any, vall, v_send, v_recv),
            ):
                rdma = pltpu.make_async_remote_copy(
                    src_ref=src_ref.at[:, pl.ds(t * HD_PER, HD_PER)],
                    dst_ref=dst.at[pl.ds(my * SKV_PER, SKV_PER), :],
                    send_sem=ssem.at[d - 1],
                    recv_sem=rsem.at[d - 1],
                    device_id=(t,),
                    device_id_type=pl.DeviceIdType.MESH,
                )
                rdma.start()
                sends.append(rdma)

        lk = pltpu.make_async_copy(
            k_any.at[:, pl.ds(my * HD_PER, HD_PER)],
            kall.at[pl.ds(my * SKV_PER, SKV_PER), :],
            loc_sem.at[0],
        )
        lv = pltpu.make_async_copy(
            v_any.at[:, pl.ds(my * HD_PER, HD_PER)],
            vall.at[pl.ds(my * SKV_PER, SKV_PER), :],
            loc_sem.at[1],
        )
        lk.start()
        lv.start()

        q32 = jnp.dot(x_ref[:, :], wq_ref[:, :], preferred_element_type=jnp.float32)
        qbuf[:, :] = q32.astype(jnp.bfloat16)

        lk.wait()
        lv.wait()
        for rdma in sends:
            rdma.wait_send()
        for d in range(1, N_DEV):
            src = lax.rem(my - d + N_DEV, N_DEV)
            for (src_ref, dst, ssem, rsem) in (
                (k_any, kall, k_send, k_recv),
                (v_any, vall, v_send, v_recv),
            ):
                recv = pltpu.make_async_remote_copy(
                    src_ref=src_ref.at[:, pl.ds(0, HD_PER)],
                    dst_ref=dst.at[pl.ds(src * SKV_PER, SKV_PER), :],
                    send_sem=ssem.at[d - 1],
                    recv_sem=rsem.at[d - 1],
                    device_id=(src,),
                    device_id_type=pl.DeviceIdType.MESH,
                )
                recv.wait_recv()

        n_loc = SQ - N_GLOB

        def head_body(h, carry):
            c0 = h * DH
            qh = qbuf[:, pl.ds(c0, DH)]
            kh = kall[:, pl.ds(c0, DH)]
            vh = vall[:, pl.ds(c0, DH)]

            qg = qh[:N_GLOB, :]
            sg = lax.dot_general(
                qg, kh, (((1,), (1,)), ((), ())),
                preferred_element_type=jnp.float32) * SCALE
            sg = sg - jnp.max(sg, axis=1, keepdims=True)
            wg = jnp.exp(sg)
            den_g = jnp.sum(wg, axis=1, keepdims=True)
            ctx_g = jnp.dot(wg.astype(jnp.bfloat16), vh,
                            preferred_element_type=jnp.float32) / den_g
            ctx_buf[:N_GLOB, pl.ds(c0, DH)] = ctx_g.astype(jnp.bfloat16)

            k_blk = kh[:N_GLOB, :]
            v_blk = vh[:N_GLOB, :]

            def tile_body(t, c):
                r0 = N_GLOB + t * LOC_TILE
                w0 = 8 * jnp.maximum(0, 31 * t - 12)
                ql = qbuf[pl.ds(r0, LOC_TILE), pl.ds(c0, DH)]
                k_win = kall[pl.ds(w0, W_WIN), pl.ds(c0, DH)]
                v_win = vall[pl.ds(w0, W_WIN), pl.ds(c0, DH)]
                sw = lax.dot_general(
                    ql, k_win, (((1,), (1,)), ((), ())),
                    preferred_element_type=jnp.float32) * SCALE
                qi = lax.broadcasted_iota(jnp.int32, (LOC_TILE, W_WIN), 0) + r0
                ki = lax.broadcasted_iota(jnp.int32, (LOC_TILE, W_WIN), 1) + w0
                mask = (jnp.abs(qi - ki) <= LOCAL_W) | (ki < N_GLOB)
                sw = jnp.where(mask, sw, -1e9)
                sb = lax.dot_general(
                    ql, k_blk, (((1,), (1,)), ((), ())),
                    preferred_element_type=jnp.float32) * SCALE
                sb = jnp.where(t > 0, sb, -1e9)
                m = jnp.maximum(jnp.max(sw, axis=1, keepdims=True),
                                jnp.max(sb, axis=1, keepdims=True))
                ww = jnp.exp(sw - m)
                wb = jnp.exp(sb - m)
                den = (jnp.sum(ww, axis=1, keepdims=True)
                       + jnp.sum(wb, axis=1, keepdims=True))
                ctx_l = (jnp.dot(ww.astype(jnp.bfloat16), v_win,
                                 preferred_element_type=jnp.float32)
                         + jnp.dot(wb.astype(jnp.bfloat16), v_blk,
                                   preferred_element_type=jnp.float32)) / den
                ctx_buf[pl.ds(r0, LOC_TILE), pl.ds(c0, DH)] = (
                    ctx_l.astype(jnp.bfloat16))
                return c

            lax.fori_loop(0, n_loc // LOC_TILE, tile_body, 0)
            return carry

        lax.fori_loop(0, HQ_PER, head_body, 0)

        partial = jnp.dot(ctx_buf[:, :], wo_ref[:, :],
                          preferred_element_type=jnp.float32)

        pbuf[:, :] = partial.astype(jnp.bfloat16)
        out_ref[:, :] = partial
        rs_rdmas = []
        for d in range(1, N_DEV):
            t = lax.rem(my + d, N_DEV)
            rdma = pltpu.make_async_remote_copy(
                src_ref=pbuf.at[pl.ds(t * QROWS, QROWS), :],
                dst_ref=rs_buf.at[d - 1],
                send_sem=rs_send.at[d - 1],
                recv_sem=rs_recv.at[d - 1],
                device_id=(t,),
                device_id_type=pl.DeviceIdType.MESH,
            )
            rdma.start()
            rs_rdmas.append(rdma)
        for d in range(1, N_DEV):
            src = lax.rem(my - d + N_DEV, N_DEV)
            recv = pltpu.make_async_remote_copy(
                src_ref=pbuf.at[pl.ds(0, QROWS), :],
                dst_ref=rs_buf.at[d - 1],
                send_sem=rs_send.at[d - 1],
                recv_sem=rs_recv.at[d - 1],
                device_id=(src,),
                device_id_type=pl.DeviceIdType.MESH,
            )
            recv.wait_recv()
        for rdma in rs_rdmas:
            rdma.wait_send()

        red = out_ref[pl.ds(my * QROWS, QROWS), :]
        for j in range(N_DEV - 1):
            red = red + rs_buf[j, :, :].astype(jnp.float32)
        ag_buf[pl.ds(my * QROWS, QROWS), :] = red.astype(jnp.bfloat16)

        ag_rdmas = []
        for d in range(1, N_DEV):
            t = lax.rem(my + d, N_DEV)
            rdma = pltpu.make_async_remote_copy(
                src_ref=ag_buf.at[pl.ds(my * QROWS, QROWS), :],
                dst_ref=ag_buf.at[pl.ds(my * QROWS, QROWS), :],
                send_sem=ag_send.at[d - 1],
                recv_sem=ag_recv.at[d - 1],
                device_id=(t,),
                device_id_type=pl.DeviceIdType.MESH,
            )
            rdma.start()
            ag_rdmas.append(rdma)
        for d in range(1, N_DEV):
            src = lax.rem(my - d + N_DEV, N_DEV)
            recv = pltpu.make_async_remote_copy(
                src_ref=ag_buf.at[pl.ds(0, QROWS), :],
                dst_ref=ag_buf.at[pl.ds(src * QROWS, QROWS), :],
                send_sem=ag_send.at[d - 1],
                recv_sem=ag_recv.at[d - 1],
                device_id=(src,),
                device_id_type=pl.DeviceIdType.MESH,
            )
            recv.wait_recv()
        for rdma in ag_rdmas:
            rdma.wait_send()

        out_ref[:, :] = ag_buf[:, :].astype(jnp.float32)

    out = pl.pallas_call(
        body,
        out_shape=jax.ShapeDtypeStruct((SQ, DM), jnp.float32),
        in_specs=[
            pl.BlockSpec(memory_space=pltpu.VMEM),
            pl.BlockSpec(memory_space=pltpu.VMEM),
            pl.BlockSpec(memory_space=pltpu.MemorySpace.HBM),
            pl.BlockSpec(memory_space=pltpu.MemorySpace.HBM),
            pl.BlockSpec(memory_space=pltpu.VMEM),
        ],
        out_specs=pl.BlockSpec(memory_space=pltpu.VMEM),
        scratch_shapes=[
            pltpu.VMEM((SKV, HD_PER), jnp.bfloat16),
            pltpu.VMEM((SKV, HD_PER), jnp.bfloat16),
            pltpu.VMEM((SQ, HD_PER), jnp.bfloat16),
            pltpu.VMEM((SQ, HD_PER), jnp.bfloat16),
            pltpu.VMEM((SQ, DM), jnp.bfloat16),
            pltpu.VMEM((N_DEV - 1, QROWS, DM), jnp.bfloat16),
            pltpu.VMEM((SQ, DM), jnp.bfloat16),
            pltpu.SemaphoreType.DMA((N_DEV - 1,)),
            pltpu.SemaphoreType.DMA((N_DEV - 1,)),
            pltpu.SemaphoreType.DMA((N_DEV - 1,)),
            pltpu.SemaphoreType.DMA((N_DEV - 1,)),
            pltpu.SemaphoreType.DMA((N_DEV - 1,)),
            pltpu.SemaphoreType.DMA((N_DEV - 1,)),
            pltpu.SemaphoreType.DMA((N_DEV - 1,)),
            pltpu.SemaphoreType.DMA((N_DEV - 1,)),
            pltpu.SemaphoreType.DMA((2,)),
        ],
        compiler_params=pltpu.CompilerParams(collective_id=0),
    )(xb, Wqb, K2, V2, Wob)
    return out.reshape(1, SQ, DM)


# device time: 188541 ns/iter; 1.2741x vs baseline; 1.0235x over previous
import jax
import jax.numpy as jnp
from jax import lax
from jax.experimental import pallas as pl
from jax.experimental.pallas import tpu as pltpu

N_DEV = 4
SQ = 1024
SKV_PER = 1024
SKV = N_DEV * SKV_PER
HQ_PER = 8
DH = 128
DM = 1024
HD_PER = HQ_PER * DH
SCALE = 0.08838834764831843
LOCAL_W = 128
N_GLOB = 32
KV_LOC = 2 * SKV_PER
LOC_TILE = 248
W_WIN = 512
QROWS = SQ // N_DEV


def kernel(x, Wq, K_ext, V_ext, Wo):
    xb = x.reshape(SQ, DM).astype(jnp.bfloat16)
    Wqb = Wq.astype(jnp.bfloat16)
    Wob = Wo.astype(jnp.bfloat16)
    K2 = K_ext.reshape(SKV_PER, 32 * DH).astype(jnp.bfloat16)
    V2 = V_ext.reshape(SKV_PER, 32 * DH).astype(jnp.bfloat16)

    def body(x_ref, wq_ref, k_any, v_any, wo_ref, out_ref,
             kall, vall, qbuf, ctx_buf, pbuf, rs_buf, ag_buf,
             k_send, v_send, k_recv, v_recv,
             rs_send, rs_recv, ag_send, ag_recv):
        my = lax.axis_index("i")

        barrier_sem = pltpu.get_barrier_semaphore()
        for d in range(1, N_DEV):
            peer = lax.rem(my + d, N_DEV)
            pl.semaphore_signal(
                barrier_sem, inc=1,
                device_id=(peer,), device_id_type=pl.DeviceIdType.MESH,
            )
        pl.semaphore_wait(barrier_sem, N_DEV - 1)

        sends = []
        for d in range(N_DEV):
            t = lax.rem(my + d, N_DEV)
            for (src_ref, dst, ssem, rsem) in (
                (k_any, kall, k_send, k_recv),
                (v_any, vall, v_send, v_recv),
            ):
                rdma = pltpu.make_async_remote_copy(
                    src_ref=src_ref.at[:, pl.ds(t * HD_PER, HD_PER)],
                    dst_ref=dst.at[pl.ds(my * SKV_PER, SKV_PER), :],
                    send_sem=ssem.at[d],
                    recv_sem=rsem.at[d],
                    device_id=(t,),
                    device_id_type=pl.DeviceIdType.MESH,
                )
                rdma.start()
                sends.append(rdma)

        q32 = jnp.dot(x_ref[:, :], wq_ref[:, :], preferred_element_type=jnp.float32)
        qbuf[:, :] = q32.astype(jnp.bfloat16)

        def wait_chunk(c):
            dc = lax.rem(my - c + N_DEV, N_DEV)
            for (dst, ssem, rsem) in ((kall, k_send, k_recv),
                                      (vall, v_send, v_recv)):
                recv = pltpu.make_async_remote_copy(
                    src_ref=k_any.at[:, pl.ds(0, HD_PER)],
                    dst_ref=dst.at[pl.ds(c * SKV_PER, SKV_PER), :],
                    send_sem=ssem.at[dc],
                    recv_sem=rsem.at[dc],
                    device_id=(my,),
                    device_id_type=pl.DeviceIdType.MESH,
                )
                recv.wait_recv()

        def tile_heads(t):
            r0 = N_GLOB + t * LOC_TILE
            w0 = max(0, r0 - LOCAL_W)

            def f(h, carry):
                c0 = h * DH
                ql = qbuf[r0:r0 + LOC_TILE, pl.ds(c0, DH)]
                k_win = kall[w0:w0 + W_WIN, pl.ds(c0, DH)]
                v_win = vall[w0:w0 + W_WIN, pl.ds(c0, DH)]
                sw = lax.dot_general(
                    ql, k_win, (((1,), (1,)), ((), ())),
                    preferred_element_type=jnp.float32) * SCALE
                qi = lax.broadcasted_iota(jnp.int32, (LOC_TILE, W_WIN), 0) + r0
                ki = lax.broadcasted_iota(jnp.int32, (LOC_TILE, W_WIN), 1) + w0
                mask = (jnp.abs(qi - ki) <= LOCAL_W) | (ki < N_GLOB)
                sw = jnp.where(mask, sw, -1e9)
                if t == 0:
                    m = jnp.max(sw, axis=1, keepdims=True)
                    ww = jnp.exp(sw - m)
                    den = jnp.sum(ww, axis=1, keepdims=True)
                    ctx_l = jnp.dot(ww.astype(jnp.bfloat16), v_win,
                                    preferred_element_type=jnp.float32) / den
                else:
                    k_blk = kall[:N_GLOB, pl.ds(c0, DH)]
                    v_blk = vall[:N_GLOB, pl.ds(c0, DH)]
                    sb = lax.dot_general(
                        ql, k_blk, (((1,), (1,)), ((), ())),
                        preferred_element_type=jnp.float32) * SCALE
                    m = jnp.maximum(jnp.max(sw, axis=1, keepdims=True),
                                    jnp.max(sb, axis=1, keepdims=True))
                    ww = jnp.exp(sw - m)
                    wb = jnp.exp(sb - m)
                    den = (jnp.sum(ww, axis=1, keepdims=True)
                           + jnp.sum(wb, axis=1, keepdims=True))
                    ctx_l = (jnp.dot(ww.astype(jnp.bfloat16), v_win,
                                     preferred_element_type=jnp.float32)
                             + jnp.dot(wb.astype(jnp.bfloat16), v_blk,
                                       preferred_element_type=jnp.float32)
                             ) / den
                ctx_buf[r0:r0 + LOC_TILE, pl.ds(c0, DH)] = (
                    ctx_l.astype(jnp.bfloat16))
                return carry

            return f

        wait_chunk(0)
        for t in (0, 1, 2):
            lax.fori_loop(0, HQ_PER, tile_heads(t), 0)
        wait_chunk(1)
        lax.fori_loop(0, HQ_PER, tile_heads(3), 0)
        wait_chunk(2)
        wait_chunk(3)
        for rdma in sends:
            rdma.wait_send()

        def global_heads(h, carry):
            c0 = h * DH
            qg = qbuf[:N_GLOB, pl.ds(c0, DH)]
            kh = kall[:, pl.ds(c0, DH)]
            vh = vall[:, pl.ds(c0, DH)]
            sg = lax.dot_general(
                qg, kh, (((1,), (1,)), ((), ())),
                preferred_element_type=jnp.float32) * SCALE
            sg = sg - jnp.max(sg, axis=1, keepdims=True)
            wg = jnp.exp(sg)
            den_g = jnp.sum(wg, axis=1, keepdims=True)
            ctx_g = jnp.dot(wg.astype(jnp.bfloat16), vh,
                            preferred_element_type=jnp.float32) / den_g
            ctx_buf[:N_GLOB, pl.ds(c0, DH)] = ctx_g.astype(jnp.bfloat16)
            return carry

        lax.fori_loop(0, HQ_PER, global_heads, 0)

        partial = jnp.dot(ctx_buf[:, :], wo_ref[:, :],
                          preferred_element_type=jnp.float32)

        pbuf[:, :] = partial.astype(jnp.bfloat16)
        out_ref[:, :] = partial
        rs_rdmas = []
        for d in range(1, N_DEV):
            t = lax.rem(my + d, N_DEV)
            rdma = pltpu.make_async_remote_copy(
                src_ref=pbuf.at[pl.ds(t * QROWS, QROWS), :],
                dst_ref=rs_buf.at[d - 1],
                send_sem=rs_send.at[d - 1],
                recv_sem=rs_recv.at[d - 1],
                device_id=(t,),
                device_id_type=pl.DeviceIdType.MESH,
            )
            rdma.start()
            rs_rdmas.append(rdma)
        for d in range(1, N_DEV):
            src = lax.rem(my - d + N_DEV, N_DEV)
            recv = pltpu.make_async_remote_copy(
                src_ref=pbuf.at[pl.ds(0, QROWS), :],
                dst_ref=rs_buf.at[d - 1],
                send_sem=rs_send.at[d - 1],
                recv_sem=rs_recv.at[d - 1],
                device_id=(src,),
                device_id_type=pl.DeviceIdType.MESH,
            )
            recv.wait_recv()
        for rdma in rs_rdmas:
            rdma.wait_send()

        red = out_ref[pl.ds(my * QROWS, QROWS), :]
        for j in range(N_DEV - 1):
            red = red + rs_buf[j, :, :].astype(jnp.float32)
        ag_buf[pl.ds(my * QROWS, QROWS), :] = red.astype(jnp.bfloat16)

        ag_rdmas = []
        for d in range(1, N_DEV):
            t = lax.rem(my + d, N_DEV)
            rdma = pltpu.make_async_remote_copy(
                src_ref=ag_buf.at[pl.ds(my * QROWS, QROWS), :],
                dst_ref=ag_buf.at[pl.ds(my * QROWS, QROWS), :],
                send_sem=ag_send.at[d - 1],
                recv_sem=ag_recv.at[d - 1],
                device_id=(t,),
                device_id_type=pl.DeviceIdType.MESH,
            )
            rdma.start()
            ag_rdmas.append(rdma)
        for d in range(1, N_DEV):
            src = lax.rem(my - d + N_DEV, N_DEV)
            recv = pltpu.make_async_remote_copy(
                src_ref=ag_buf.at[pl.ds(0, QROWS), :],
                dst_ref=ag_buf.at[pl.ds(src * QROWS, QROWS), :],
                send_sem=ag_send.at[d - 1],
                recv_sem=ag_recv.at[d - 1],
                device_id=(src,),
                device_id_type=pl.DeviceIdType.MESH,
            )
            recv.wait_recv()
        for rdma in ag_rdmas:
            rdma.wait_send()

        out_ref[:, :] = ag_buf[:, :].astype(jnp.float32)

    out = pl.pallas_call(
        body,
        out_shape=jax.ShapeDtypeStruct((SQ, DM), jnp.float32),
        in_specs=[
            pl.BlockSpec(memory_space=pltpu.VMEM),
            pl.BlockSpec(memory_space=pltpu.VMEM),
            pl.BlockSpec(memory_space=pltpu.MemorySpace.HBM),
            pl.BlockSpec(memory_space=pltpu.MemorySpace.HBM),
            pl.BlockSpec(memory_space=pltpu.VMEM),
        ],
        out_specs=pl.BlockSpec(memory_space=pltpu.VMEM),
        scratch_shapes=[
            pltpu.VMEM((SKV, HD_PER), jnp.bfloat16),
            pltpu.VMEM((SKV, HD_PER), jnp.bfloat16),
            pltpu.VMEM((SQ, HD_PER), jnp.bfloat16),
            pltpu.VMEM((SQ, HD_PER), jnp.bfloat16),
            pltpu.VMEM((SQ, DM), jnp.bfloat16),
            pltpu.VMEM((N_DEV - 1, QROWS, DM), jnp.bfloat16),
            pltpu.VMEM((SQ, DM), jnp.bfloat16),
            pltpu.SemaphoreType.DMA((N_DEV,)),
            pltpu.SemaphoreType.DMA((N_DEV,)),
            pltpu.SemaphoreType.DMA((N_DEV,)),
            pltpu.SemaphoreType.DMA((N_DEV,)),
            pltpu.SemaphoreType.DMA((N_DEV - 1,)),
            pltpu.SemaphoreType.DMA((N_DEV - 1,)),
            pltpu.SemaphoreType.DMA((N_DEV - 1,)),
            pltpu.SemaphoreType.DMA((N_DEV - 1,)),
        ],
        compiler_params=pltpu.CompilerParams(collective_id=0),
    )(xb, Wqb, K2, V2, Wob)
    return out.reshape(1, SQ, DM)
